# Initial kernel scaffold; baseline (speedup 1.0000x reference)
import jax
import jax.numpy as jnp
from jax.experimental import pallas as pl


def kernel(x, edge_index, W_self, W_neigh, b):
    raise NotImplementedError("write your pallas kernel here")


# speedup vs baseline: 3.8300x; 3.8300x over previous
import functools

import jax
import jax.numpy as jnp
from jax import lax
from jax.experimental import pallas as pl
from jax.experimental.pallas import tpu as pltpu
from jax.experimental.pallas import tpu_sc as plsc

N_NODES = 10000
N_EDGES = 320000
D = 128

NC = 2
NS = 16
NW = NC * NS
GROUP = 128
ROWS_PER_TILE = 80
E_PAD = NW * ROWS_PER_TILE * GROUP
R_TOTAL = E_PAD // GROUP
ZCHUNK = 640
AGG_ROWS = NS * ZCHUNK
CH = 128
NSTRIPE = ZCHUNK // CH


def _sc_aggregate(x, src2d, dst2d, sidx, zfeat, ones128):
    mesh = plsc.VectorSubcoreMesh(core_axis_name="c", subcore_axis_name="s")

    @functools.partial(
        pl.kernel,
        mesh=mesh,
        out_type=[
            jax.ShapeDtypeStruct((NC * AGG_ROWS, D), jnp.float32),
            jax.ShapeDtypeStruct((NC * AGG_ROWS, D), jnp.float32),
        ],
        scratch_types=[
            pltpu.VMEM((8, GROUP), jnp.int32),
            pltpu.VMEM((8, GROUP), jnp.int32),
            pltpu.VMEM((8, GROUP), jnp.int32),
            pltpu.VMEM((GROUP, D), jnp.float32),
            pltpu.VMEM_SHARED((AGG_ROWS, D), jnp.float32),
            pltpu.SemaphoreType.DMA,
        ],
    )
    def k(x_hbm, src_hbm, dst_hbm, sidx_hbm, zf_hbm, ones_hbm,
          agg_hbm, deg_hbm,
          src_v, dst_v, sidx_v, rows_v, agg_s, sem):
        cid = lax.axis_index("c")
        sid = lax.axis_index("s")
        wid = sid * NC + cid
        row_base = wid * ROWS_PER_TILE
        obase = cid * AGG_ROWS + sid * ZCHUNK

        pltpu.sync_copy(zf_hbm, rows_v)
        pltpu.sync_copy(sidx_hbm.at[sid], sidx_v)
        for i in range(NSTRIPE):
            pltpu.sync_copy(rows_v, agg_s.at[sidx_v.at[i]])
        plsc.subcore_barrier()

        def body(rb, carry):
            row = pl.multiple_of(row_base + rb * 8, 8)
            pltpu.sync_copy(src_hbm.at[pl.ds(row, 8)], src_v)
            pltpu.sync_copy(dst_hbm.at[pl.ds(row, 8)], dst_v)
            for j in range(8):
                pltpu.async_copy(x_hbm.at[src_v.at[j]], rows_v, sem).wait()
                pltpu.sync_copy(rows_v, agg_s.at[dst_v.at[j]], add=True)
            return carry

        lax.fori_loop(0, ROWS_PER_TILE // 8, body, 0)
        plsc.subcore_barrier()

        for i in range(NSTRIPE):
            pltpu.sync_copy(agg_s.at[sidx_v.at[i]], rows_v)
            pltpu.sync_copy(rows_v, agg_hbm.at[pl.ds(obase + i * CH, CH)])
        pltpu.sync_copy(zf_hbm, rows_v)
        for i in range(NSTRIPE):
            pltpu.sync_copy(rows_v, agg_s.at[sidx_v.at[i]])
        pltpu.sync_copy(ones_hbm, rows_v)
        plsc.subcore_barrier()

        def body2(rb, carry):
            row = pl.multiple_of(row_base + rb * 8, 8)
            pltpu.sync_copy(dst_hbm.at[pl.ds(row, 8)], dst_v)
            for j in range(8):
                pltpu.sync_copy(rows_v, agg_s.at[dst_v.at[j]], add=True)
            return carry

        lax.fori_loop(0, ROWS_PER_TILE // 8, body2, 0)
        plsc.subcore_barrier()

        for i in range(NSTRIPE):
            pltpu.sync_copy(agg_s.at[sidx_v.at[i]], rows_v)
            pltpu.sync_copy(rows_v, deg_hbm.at[pl.ds(obase + i * CH, CH)])

    return k(x, src2d, dst2d, sidx, zfeat, ones128)


BN = 1000


def _tc_body(x_ref, a0_ref, a1_ref, d0_ref, d1_ref, ws_ref, wn_ref, b_ref,
             o_ref):
    deg = jnp.maximum(d0_ref[...] + d1_ref[...], 1.0)
    h = (a0_ref[...] + a1_ref[...]) / deg
    acc = jnp.dot(x_ref[...], ws_ref[...], preferred_element_type=jnp.float32)
    acc = acc + jnp.dot(h, wn_ref[...], preferred_element_type=jnp.float32)
    o_ref[...] = jnp.maximum(acc + b_ref[...], 0.0)


def _tc_combine(x, a0, a1, d0, d1, W_self, W_neigh, b2d):
    grid = (N_NODES // BN,)
    return pl.pallas_call(
        _tc_body,
        grid=grid,
        in_specs=[
            pl.BlockSpec((BN, D), lambda i: (i, 0)),
            pl.BlockSpec((BN, D), lambda i: (i, 0)),
            pl.BlockSpec((BN, D), lambda i: (i, 0)),
            pl.BlockSpec((BN, 1), lambda i: (i, 0)),
            pl.BlockSpec((BN, 1), lambda i: (i, 0)),
            pl.BlockSpec((D, D), lambda i: (0, 0)),
            pl.BlockSpec((D, D), lambda i: (0, 0)),
            pl.BlockSpec((1, D), lambda i: (0, 0)),
        ],
        out_specs=pl.BlockSpec((BN, D), lambda i: (i, 0)),
        out_shape=jax.ShapeDtypeStruct((N_NODES, D), jnp.float32),
    )(x, a0, a1, d0, d1, W_self, W_neigh, b2d)


def kernel(x, edge_index, W_self, W_neigh, b):
    src = edge_index[0].astype(jnp.int32)
    dst = edge_index[1].astype(jnp.int32)
    pad = E_PAD - N_EDGES
    src_p = jnp.concatenate([src, jnp.zeros((pad,), jnp.int32)])
    dst_p = jnp.concatenate([dst, jnp.full((pad,), N_NODES, jnp.int32)])
    src2d = src_p.reshape(R_TOTAL, GROUP)
    dst2d = dst_p.reshape(R_TOTAL, GROUP)
    zfeat = jnp.zeros((GROUP, D), jnp.float32)
    ones128 = jnp.ones((GROUP, D), jnp.float32)
    sidx = (jnp.arange(NS, dtype=jnp.int32)[:, None, None] * ZCHUNK
            + jnp.minimum(jnp.arange(8, dtype=jnp.int32), NSTRIPE - 1)[None, :, None] * CH
            + jnp.arange(GROUP, dtype=jnp.int32)[None, None, :])

    agg, deg = _sc_aggregate(x, src2d, dst2d, sidx, zfeat, ones128)
    a0 = agg[:N_NODES]
    a1 = agg[AGG_ROWS:AGG_ROWS + N_NODES]
    d0 = deg[:N_NODES, 0:1]
    d1 = deg[AGG_ROWS:AGG_ROWS + N_NODES, 0:1]
    return _tc_combine(x, a0, a1, d0, d1, W_self, W_neigh, b.reshape(1, D))


# speedup vs baseline: 4.1048x; 1.0718x over previous
import functools

import jax
import jax.numpy as jnp
from jax import lax
from jax.experimental import pallas as pl
from jax.experimental.pallas import tpu as pltpu
from jax.experimental.pallas import tpu_sc as plsc

N_NODES = 10000
N_EDGES = 320000
D = 128

NC = 2
NS = 16
NW = NC * NS
GROUP = 128
ROWS_PER_TILE = 80
E_PAD = NW * ROWS_PER_TILE * GROUP
R_TOTAL = E_PAD // GROUP
ZCHUNK = 640
AGG_ROWS = NS * ZCHUNK
CH = 128
NSTRIPE = ZCHUNK // CH


def _sc_aggregate(x, src2d, dst2d, sidx, zfeat, ones128):
    mesh = plsc.VectorSubcoreMesh(core_axis_name="c", subcore_axis_name="s")

    @functools.partial(
        pl.kernel,
        mesh=mesh,
        out_type=[
            jax.ShapeDtypeStruct((NC * AGG_ROWS, D), jnp.float32),
            jax.ShapeDtypeStruct((NC * AGG_ROWS, D), jnp.float32),
        ],
        scratch_types=[
            pltpu.VMEM((8, GROUP), jnp.int32),
            pltpu.VMEM((8, GROUP), jnp.int32),
            pltpu.VMEM((8, GROUP), jnp.int32),
            pltpu.VMEM((GROUP, D), jnp.float32),
            pltpu.VMEM((GROUP, D), jnp.float32),
            pltpu.VMEM_SHARED((AGG_ROWS, D), jnp.float32),
            pltpu.SemaphoreType.DMA,
            pltpu.SemaphoreType.DMA,
        ],
    )
    def k(x_hbm, src_hbm, dst_hbm, sidx_hbm, zf_hbm, ones_hbm,
          agg_hbm, deg_hbm,
          src_v, dst_v, sidx_v, rows_v, rows2_v, agg_s, sem, sem2):
        cid = lax.axis_index("c")
        sid = lax.axis_index("s")
        wid = sid * NC + cid
        row_base = wid * ROWS_PER_TILE
        obase = cid * AGG_ROWS + sid * ZCHUNK

        pltpu.sync_copy(zf_hbm, rows_v)
        pltpu.sync_copy(sidx_hbm.at[sid], sidx_v)
        for i in range(NSTRIPE):
            pltpu.sync_copy(rows_v, agg_s.at[sidx_v.at[i]])
        plsc.subcore_barrier()

        bufs = (rows_v, rows2_v)
        sems = (sem, sem2)

        def body(rb, carry):
            row = pl.multiple_of(row_base + rb * 8, 8)
            pltpu.sync_copy(src_hbm.at[pl.ds(row, 8)], src_v)
            pltpu.sync_copy(dst_hbm.at[pl.ds(row, 8)], dst_v)
            pltpu.async_copy(x_hbm.at[src_v.at[0]], rows_v, sem)
            for j in range(8):
                b, s = bufs[j % 2], sems[j % 2]
                pltpu.make_async_copy(x_hbm.at[src_v.at[j]], b, s).wait()
                if j < 7:
                    pltpu.async_copy(x_hbm.at[src_v.at[j + 1]],
                                     bufs[(j + 1) % 2], sems[(j + 1) % 2])
                pltpu.sync_copy(b, agg_s.at[dst_v.at[j]], add=True)
            return carry

        lax.fori_loop(0, ROWS_PER_TILE // 8, body, 0)
        plsc.subcore_barrier()

        for i in range(NSTRIPE):
            pltpu.sync_copy(agg_s.at[sidx_v.at[i]], rows_v)
            pltpu.sync_copy(rows_v, agg_hbm.at[pl.ds(obase + i * CH, CH)])
        pltpu.sync_copy(zf_hbm, rows_v)
        for i in range(NSTRIPE):
            pltpu.sync_copy(rows_v, agg_s.at[sidx_v.at[i]])
        pltpu.sync_copy(ones_hbm, rows_v)
        plsc.subcore_barrier()

        def body2(rb, carry):
            row = pl.multiple_of(row_base + rb * 8, 8)
            pltpu.sync_copy(dst_hbm.at[pl.ds(row, 8)], dst_v)
            for j in range(8):
                pltpu.sync_copy(rows_v, agg_s.at[dst_v.at[j]], add=True)
            return carry

        lax.fori_loop(0, ROWS_PER_TILE // 8, body2, 0)
        plsc.subcore_barrier()

        for i in range(NSTRIPE):
            pltpu.sync_copy(agg_s.at[sidx_v.at[i]], rows_v)
            pltpu.sync_copy(rows_v, deg_hbm.at[pl.ds(obase + i * CH, CH)])

    return k(x, src2d, dst2d, sidx, zfeat, ones128)


BN = 1000


def _tc_body(x_ref, a0_ref, a1_ref, d0_ref, d1_ref, ws_ref, wn_ref, b_ref,
             o_ref):
    deg = jnp.maximum(d0_ref[...] + d1_ref[...], 1.0)
    h = (a0_ref[...] + a1_ref[...]) / deg
    acc = jnp.dot(x_ref[...], ws_ref[...], preferred_element_type=jnp.float32)
    acc = acc + jnp.dot(h, wn_ref[...], preferred_element_type=jnp.float32)
    o_ref[...] = jnp.maximum(acc + b_ref[...], 0.0)


def _tc_combine(x, a0, a1, d0, d1, W_self, W_neigh, b2d):
    grid = (N_NODES // BN,)
    return pl.pallas_call(
        _tc_body,
        grid=grid,
        in_specs=[
            pl.BlockSpec((BN, D), lambda i: (i, 0)),
            pl.BlockSpec((BN, D), lambda i: (i, 0)),
            pl.BlockSpec((BN, D), lambda i: (i, 0)),
            pl.BlockSpec((BN, 1), lambda i: (i, 0)),
            pl.BlockSpec((BN, 1), lambda i: (i, 0)),
            pl.BlockSpec((D, D), lambda i: (0, 0)),
            pl.BlockSpec((D, D), lambda i: (0, 0)),
            pl.BlockSpec((1, D), lambda i: (0, 0)),
        ],
        out_specs=pl.BlockSpec((BN, D), lambda i: (i, 0)),
        out_shape=jax.ShapeDtypeStruct((N_NODES, D), jnp.float32),
    )(x, a0, a1, d0, d1, W_self, W_neigh, b2d)


def kernel(x, edge_index, W_self, W_neigh, b):
    src = edge_index[0].astype(jnp.int32)
    dst = edge_index[1].astype(jnp.int32)
    pad = E_PAD - N_EDGES
    src_p = jnp.concatenate([src, jnp.zeros((pad,), jnp.int32)])
    dst_p = jnp.concatenate([dst, jnp.full((pad,), N_NODES, jnp.int32)])
    src2d = src_p.reshape(R_TOTAL, GROUP)
    dst2d = dst_p.reshape(R_TOTAL, GROUP)
    zfeat = jnp.zeros((GROUP, D), jnp.float32)
    ones128 = jnp.ones((GROUP, D), jnp.float32)
    sidx = (jnp.arange(NS, dtype=jnp.int32)[:, None, None] * ZCHUNK
            + jnp.minimum(jnp.arange(8, dtype=jnp.int32), NSTRIPE - 1)[None, :, None] * CH
            + jnp.arange(GROUP, dtype=jnp.int32)[None, None, :])

    agg, deg = _sc_aggregate(x, src2d, dst2d, sidx, zfeat, ones128)
    a0 = agg[:N_NODES]
    a1 = agg[AGG_ROWS:AGG_ROWS + N_NODES]
    d0 = deg[:N_NODES, 0:1]
    d1 = deg[AGG_ROWS:AGG_ROWS + N_NODES, 0:1]
    return _tc_combine(x, a0, a1, d0, d1, W_self, W_neigh, b.reshape(1, D))
